# baseline (speedup 1.0000x reference)
import functools

import jax
import jax.numpy as jnp
from jax import lax
from jax.experimental import pallas as pl
from jax.experimental.pallas import tpu as pltpu
from jax.experimental.pallas import tpu_sc as plsc

B, T, C, H, W = 4, 32, 3, 224, 224
S = max(1, T // 4)

_NC = 2
_NS = 16
_NW = _NC * _NS
_FPW = (B * T) // _NW
_NCH = _FPW * C

_mesh = plsc.VectorSubcoreMesh(core_axis_name="c", subcore_axis_name="s")


@functools.partial(
    pl.kernel,
    out_type=jax.ShapeDtypeStruct((B, T, C, H, W), jnp.float32),
    mesh=_mesh,
    scratch_types=[
        pltpu.VMEM((H, W), jnp.float32),
        pltpu.VMEM((H, W), jnp.float32),
        pltpu.SemaphoreType.DMA,
        pltpu.SemaphoreType.DMA,
        pltpu.SemaphoreType.DMA,
        pltpu.SemaphoreType.DMA,
    ],
)
def _fast_copy_sc(frames_hbm, out_hbm, buf0, buf1, si0, si1, so0, so1):
    wid = lax.axis_index("s") * _NC + lax.axis_index("c")
    bufs = (buf0, buf1)
    sin = (si0, si1)
    sout = (so0, so1)

    def chunk_ref(ref, q):
        f = wid * _FPW + q // C
        return ref.at[f // T, f % T, q % C]

    def start_in(q):
        return pltpu.async_copy(chunk_ref(frames_hbm, q), bufs[q % 2], sin[q % 2])

    def start_out(q):
        return pltpu.async_copy(bufs[q % 2], chunk_ref(out_hbm, q), sout[q % 2])

    ins = {0: start_in(0), 1: start_in(1)}
    outs = {}
    for q in range(_NCH):
        ins[q].wait()
        outs[q] = start_out(q)
        if q + 2 < _NCH:
            outs[q].wait()
            ins[q + 2] = start_in(q + 2)
    outs[_NCH - 2].wait()
    outs[_NCH - 1].wait()


def _slow_body(x_ref, o_ref):
    o_ref[...] = x_ref[...]


def _slow_gather_tc(frames):
    return pl.pallas_call(
        _slow_body,
        grid=(B, S),
        in_specs=[pl.BlockSpec((1, 1, C, H, W),
                               lambda i, j: (i, (j * (T - 1)) // (S - 1), 0, 0, 0))],
        out_specs=pl.BlockSpec((1, 1, C, H, W), lambda i, j: (i, j, 0, 0, 0)),
        out_shape=jax.ShapeDtypeStruct((B, S, C, H, W), jnp.float32),
    )(frames)


def kernel(frames):
    return (_slow_gather_tc(frames), _fast_copy_sc(frames))


# speedup vs baseline: 1.0654x; 1.0654x over previous
import functools

import jax
import jax.numpy as jnp
from jax import lax
from jax.experimental import pallas as pl
from jax.experimental.pallas import tpu as pltpu
from jax.experimental.pallas import tpu_sc as plsc

B, T, C, H, W = 4, 32, 3, 224, 224
S = max(1, T // 4)

_NC = 2
_NS = 16

_mesh = plsc.VectorSubcoreMesh(core_axis_name="c", subcore_axis_name="s")


@functools.partial(
    pl.kernel,
    out_type=jax.ShapeDtypeStruct((B, S, C, H, W), jnp.float32),
    mesh=_mesh,
    scratch_types=[
        pltpu.VMEM((H, W), jnp.float32),
        pltpu.VMEM((H, W), jnp.float32),
        pltpu.SemaphoreType.DMA,
        pltpu.SemaphoreType.DMA,
        pltpu.SemaphoreType.DMA,
        pltpu.SemaphoreType.DMA,
    ],
)
def _slow_gather(frames_hbm, out_hbm, buf0, buf1, si0, si1, so0, so1):
    wid = lax.axis_index("s") * _NC + lax.axis_index("c")
    b = wid // S
    s = wid % S
    src_t = (s * (T - 1)) // (S - 1)
    in0 = pltpu.async_copy(frames_hbm.at[b, src_t, 0], buf0, si0)
    in1 = pltpu.async_copy(frames_hbm.at[b, src_t, 1], buf1, si1)
    in0.wait()
    out0 = pltpu.async_copy(buf0, out_hbm.at[b, s, 0], so0)
    in1.wait()
    out1 = pltpu.async_copy(buf1, out_hbm.at[b, s, 1], so1)
    out0.wait()
    in2 = pltpu.async_copy(frames_hbm.at[b, src_t, 2], buf0, si0)
    in2.wait()
    out2 = pltpu.async_copy(buf0, out_hbm.at[b, s, 2], so0)
    out1.wait()
    out2.wait()


def _alias_body(x_hbm, o_hbm):
    pass


def _fast_alias(frames):
    return pl.pallas_call(
        _alias_body,
        in_specs=[pl.BlockSpec(memory_space=pltpu.MemorySpace.HBM)],
        out_specs=pl.BlockSpec(memory_space=pltpu.MemorySpace.HBM),
        out_shape=jax.ShapeDtypeStruct((B, T, C, H, W), jnp.float32),
        input_output_aliases={0: 0},
    )(frames)


def kernel(frames):
    return (_slow_gather(frames), _fast_alias(frames))


# speedup vs baseline: 1.0978x; 1.0304x over previous
import functools

import jax
import jax.numpy as jnp
from jax import lax
from jax.experimental import pallas as pl
from jax.experimental.pallas import tpu as pltpu
from jax.experimental.pallas import tpu_sc as plsc

B, T, C, H, W = 4, 32, 3, 224, 224
S = max(1, T // 4)

_NC = 2
_NS = 16

_mesh = plsc.VectorSubcoreMesh(core_axis_name="c", subcore_axis_name="s")


@functools.partial(
    pl.kernel,
    out_type=jax.ShapeDtypeStruct((B, S, C, H, W), jnp.float32),
    mesh=_mesh,
    scratch_types=[
        pltpu.VMEM((H, W), jnp.float32),
        pltpu.VMEM((H, W), jnp.float32),
        pltpu.SemaphoreType.DMA,
        pltpu.SemaphoreType.DMA,
        pltpu.SemaphoreType.DMA,
        pltpu.SemaphoreType.DMA,
    ],
)
def _slow_gather(frames_hbm, out_hbm, buf0, buf1, si0, si1, so0, so1):
    wid = lax.axis_index("s") * _NC + lax.axis_index("c")
    b = wid // S
    s = wid % S
    src_t = (s * (T - 1)) // (S - 1)
    in0 = pltpu.async_copy(frames_hbm.at[b, src_t, 0], buf0, si0)
    in1 = pltpu.async_copy(frames_hbm.at[b, src_t, 1], buf1, si1)
    in0.wait()
    out0 = pltpu.async_copy(buf0, out_hbm.at[b, s, 0], so0)
    in1.wait()
    out1 = pltpu.async_copy(buf1, out_hbm.at[b, s, 1], so1)
    out0.wait()
    in2 = pltpu.async_copy(frames_hbm.at[b, src_t, 2], buf0, si0)
    in2.wait()
    out2 = pltpu.async_copy(buf0, out_hbm.at[b, s, 2], so0)
    out1.wait()
    out2.wait()


def _fast_copy_body(x_ref, o_ref):
    o_ref[...] = x_ref[...]


_TBLK = 16


def _fast_copy(frames):
    return pl.pallas_call(
        _fast_copy_body,
        grid=(B, T // _TBLK),
        in_specs=[pl.BlockSpec((1, _TBLK, C, H, W), lambda i, j: (i, j, 0, 0, 0))],
        out_specs=pl.BlockSpec((1, _TBLK, C, H, W), lambda i, j: (i, j, 0, 0, 0)),
        out_shape=jax.ShapeDtypeStruct((B, T, C, H, W), jnp.float32),
    )(frames)


def kernel(frames):
    return (_slow_gather(frames), _fast_copy(frames))
